# baseline (speedup 1.0000x reference)
import dataclasses
import functools
import math

import jax
import jax.numpy as jnp
from jax.experimental import pallas as pl
from jax.experimental.pallas import tpu as pltpu
from jax.experimental.pallas import tpu_sc as plsc

_SCALE = 64.0
_MARGIN = 0.5
_COS_M = math.cos(_MARGIN)
_SIN_M = math.sin(_MARGIN)
_LOG2E = math.log2(math.e)
_SE = _SCALE * _LOG2E

_R = 8
_CW = 2048
_GW = 128


def _sc_gather(flat16, row16):
    n_idx = row16.shape[0]
    idx2 = row16.reshape(1, n_idx)

    cp = pltpu.CompilerParams()
    if "needs_layout_passes" in pltpu.CompilerParams.__dataclass_fields__:
        cp = dataclasses.replace(cp, needs_layout_passes=False)

    @functools.partial(
        pl.kernel,
        out_type=jax.ShapeDtypeStruct((n_idx, 128), flat16.dtype),
        mesh=plsc.VectorSubcoreMesh(core_axis_name="c", subcore_axis_name="s"),
        compiler_params=cp,
    )
    def gather_kernel(x_hbm, i_hbm, o_hbm):
        def body(i_vmem, o_vmem):
            pltpu.sync_copy(x_hbm.at[i_vmem.at[0]], o_vmem)

        pltpu.emit_pipeline(
            body,
            grid=(n_idx // _GW,),
            in_specs=[pl.BlockSpec((1, _GW), index_map=lambda i: (0, i))],
            out_specs=[pl.BlockSpec((_GW, 128), index_map=lambda i: (i, 0))],
            core_axis_name="s",
            dimension_semantics=(pltpu.PARALLEL,),
        )(i_hbm, o_hbm)

    return gather_kernel(flat16, idx2)


def _loss_body(lane_ref, x16_ref, x_ref, out_ref, *, n_rows, n_cols):
    i = pl.program_id(0)

    n_full = n_cols // _CW
    tail = n_cols - n_full * _CW

    def tree128(v):
        parts = [v[:, k * 128:(k + 1) * 128] for k in range(v.shape[1] // 128)]
        while len(parts) > 1:
            half = (len(parts) + 1) // 2
            parts = [
                parts[m] + parts[m + half] if m + half < len(parts) else parts[m]
                for m in range(half)
            ]
        return parts[0]

    def col_body(j, acc):
        chunk = x_ref[:, pl.ds(j * _CW, _CW)]
        xc = jnp.clip(chunk, -1.0, 1.0)
        return acc + tree128(jnp.exp2(xc * _SE - _SE))

    acc = jax.lax.fori_loop(
        0, n_full, col_body, jnp.zeros((_R, 128), jnp.float32), unroll=2
    )
    s0 = jnp.sum(acc, axis=1)
    if tail:
        xc = jnp.clip(x_ref[:, pl.ds(n_full * _CW, tail)], -1.0, 1.0)
        s0 = s0 + jnp.sum(jnp.exp2(xc * _SE - _SE), axis=1)

    onehot = jax.lax.broadcasted_iota(jnp.int32, (_R, 128), 1) == lane_ref[...]
    c = jnp.sum(jnp.where(onehot, jnp.clip(x16_ref[...], -1.0, 1.0), 0.0), axis=1)

    t_new = _SCALE * (c * _COS_M - _SIN_M * jnp.sqrt(jnp.maximum(1.0 - c * c, 0.0)))
    e_old = jnp.exp2(c * _SE - _SE)
    e_new = jnp.exp(t_new - _SCALE)
    s = s0 - e_old + e_new
    row_loss = _SCALE + jnp.log(s) - t_new

    @pl.when(i == 0)
    def _():
        out_ref[0, 0] = 0.0

    out_ref[0, 0] += jnp.sum(row_loss) * (1.0 / n_rows)


@jax.jit
def kernel(logits, labels):
    n_rows, n_cols = logits.shape
    labels = labels.astype(jnp.int32)

    flat128 = logits.reshape(n_rows * n_cols // 128, 128)
    fidx = jnp.arange(n_rows, dtype=jnp.int32) * n_cols + labels
    x128 = _sc_gather(flat128, fidx // 128)
    lane128 = (fidx % 128).reshape(n_rows, 1)

    out = pl.pallas_call(
        functools.partial(_loss_body, n_rows=n_rows, n_cols=n_cols),
        grid=(n_rows // _R,),
        in_specs=[
            pl.BlockSpec((_R, 1), lambda i: (i, 0)),
            pl.BlockSpec((_R, 128), lambda i: (i, 0)),
            pl.BlockSpec((_R, n_cols), lambda i: (i, 0)),
        ],
        out_specs=pl.BlockSpec((1, 1), lambda i: (0, 0), memory_space=pltpu.SMEM),
        out_shape=jax.ShapeDtypeStruct((1, 1), jnp.float32),
    )(lane128, x128, logits)
    return out[0, 0]


# speedup vs baseline: 1.9536x; 1.9536x over previous
import functools
import math

import jax
import jax.numpy as jnp
from jax.experimental import pallas as pl
from jax.experimental.pallas import tpu as pltpu
from jax.experimental.pallas import tpu_sc as plsc

_SCALE = 64.0
_MARGIN = 0.5
_COS_M = math.cos(_MARGIN)
_SIN_M = math.sin(_MARGIN)
_LOG2E = math.log2(math.e)
_SE = _SCALE * _LOG2E

_R = 8
_CW = 2048


def _sc_gather_rows(logits, labels):
    n_rows, n_cols = logits.shape

    @functools.partial(
        pl.kernel,
        out_type=jax.ShapeDtypeStruct((n_rows, 128), logits.dtype),
        mesh=plsc.ScalarSubcoreMesh(axis_name="c", num_cores=2),
        scratch_types=[
            pltpu.SMEM((n_rows,), jnp.int32),
            pltpu.SemaphoreType.DMA,
            pltpu.SemaphoreType.DMA,
        ],
    )
    def gather_kernel(x_hbm, l_hbm, o_hbm, l_smem, sem_l, sem_d):
        core = jax.lax.axis_index("c")
        pltpu.async_copy(l_hbm, l_smem, sem_l).wait()
        half = n_rows // 2
        base = core * half

        @pl.loop(0, half)
        def _(i):
            r = base + i
            st = (l_smem[r] // 128) * 128
            pltpu.async_copy(x_hbm.at[r, pl.ds(st, 128)], o_hbm.at[r], sem_d)

        @pl.loop(0, half)
        def _(i):
            r = base + i
            st = (l_smem[r] // 128) * 128
            pltpu.make_async_copy(
                x_hbm.at[r, pl.ds(st, 128)], o_hbm.at[r], sem_d
            ).wait()

    return gather_kernel(logits, labels)


def _loss_body(lane_ref, x128_ref, x_ref, out_ref, *, n_rows, n_cols):
    i = pl.program_id(0)

    n_full = n_cols // _CW
    tail = n_cols - n_full * _CW

    def tree128(v):
        parts = [v[:, k * 128:(k + 1) * 128] for k in range(v.shape[1] // 128)]
        while len(parts) > 1:
            half = (len(parts) + 1) // 2
            parts = [
                parts[m] + parts[m + half] if m + half < len(parts) else parts[m]
                for m in range(half)
            ]
        return parts[0]

    def col_body(j, acc):
        chunk = x_ref[:, pl.ds(j * _CW, _CW)]
        xc = jnp.clip(chunk, -1.0, 1.0)
        return acc + tree128(jnp.exp2(xc * _SE - _SE))

    acc = jax.lax.fori_loop(
        0, n_full, col_body, jnp.zeros((_R, 128), jnp.float32), unroll=2
    )
    s0 = jnp.sum(acc, axis=1)
    if tail:
        xc = jnp.clip(x_ref[:, pl.ds(n_full * _CW, tail)], -1.0, 1.0)
        s0 = s0 + jnp.sum(jnp.exp2(xc * _SE - _SE), axis=1)

    onehot = jax.lax.broadcasted_iota(jnp.int32, (_R, 128), 1) == lane_ref[...]
    c = jnp.sum(jnp.where(onehot, jnp.clip(x128_ref[...], -1.0, 1.0), 0.0), axis=1)

    t_new = _SCALE * (c * _COS_M - _SIN_M * jnp.sqrt(jnp.maximum(1.0 - c * c, 0.0)))
    e_old = jnp.exp2(c * _SE - _SE)
    e_new = jnp.exp(t_new - _SCALE)
    s = s0 - e_old + e_new
    row_loss = _SCALE + jnp.log(s) - t_new

    @pl.when(i == 0)
    def _():
        out_ref[0, 0] = 0.0

    out_ref[0, 0] += jnp.sum(row_loss) * (1.0 / n_rows)


@jax.jit
def kernel(logits, labels):
    n_rows, n_cols = logits.shape
    labels = labels.astype(jnp.int32)

    x128 = _sc_gather_rows(logits, labels)
    lane128 = (labels % 128).reshape(n_rows, 1)

    out = pl.pallas_call(
        functools.partial(_loss_body, n_rows=n_rows, n_cols=n_cols),
        grid=(n_rows // _R,),
        in_specs=[
            pl.BlockSpec((_R, 1), lambda i: (i, 0)),
            pl.BlockSpec((_R, 128), lambda i: (i, 0)),
            pl.BlockSpec((_R, n_cols), lambda i: (i, 0)),
        ],
        out_specs=pl.BlockSpec((1, 1), lambda i: (0, 0), memory_space=pltpu.SMEM),
        out_shape=jax.ShapeDtypeStruct((1, 1), jnp.float32),
    )(lane128, x128, logits)
    return out[0, 0]


# speedup vs baseline: 2.1453x; 1.0981x over previous
import functools
import math

import jax
import jax.numpy as jnp
from jax.experimental import pallas as pl
from jax.experimental.pallas import tpu as pltpu
from jax.experimental.pallas import tpu_sc as plsc

_SCALE = 64.0
_MARGIN = 0.5
_COS_M = math.cos(_MARGIN)
_SIN_M = math.sin(_MARGIN)
_LOG2E = math.log2(math.e)
_SE = _SCALE * _LOG2E

_R = 16
_CW = 2048


def _sc_gather_rows(logits, labels):
    n_rows, n_cols = logits.shape

    @functools.partial(
        pl.kernel,
        out_type=jax.ShapeDtypeStruct((n_rows, 128), logits.dtype),
        mesh=plsc.ScalarSubcoreMesh(axis_name="c", num_cores=2),
        scratch_types=[
            pltpu.SMEM((n_rows,), jnp.int32),
            pltpu.SemaphoreType.DMA,
            pltpu.SemaphoreType.DMA,
        ],
    )
    def gather_kernel(x_hbm, l_hbm, o_hbm, l_smem, sem_l, sem_d):
        core = jax.lax.axis_index("c")
        pltpu.async_copy(l_hbm, l_smem, sem_l).wait()
        half = n_rows // 2
        base = core * half

        @pl.loop(0, half)
        def _(i):
            r = base + i
            st = (l_smem[r] // 128) * 128
            pltpu.async_copy(x_hbm.at[r, pl.ds(st, 128)], o_hbm.at[r], sem_d)

        @pl.loop(0, half)
        def _(i):
            r = base + i
            st = (l_smem[r] // 128) * 128
            pltpu.make_async_copy(
                x_hbm.at[r, pl.ds(st, 128)], o_hbm.at[r], sem_d
            ).wait()

    return gather_kernel(logits, labels)


def _loss_body(lane_ref, x128_ref, x_ref, out_ref, *, n_rows, n_cols):
    i = pl.program_id(0)

    n_full = n_cols // _CW
    tail = n_cols - n_full * _CW

    def tree128(v):
        parts = [v[:, k * 128:(k + 1) * 128] for k in range(v.shape[1] // 128)]
        while len(parts) > 1:
            half = (len(parts) + 1) // 2
            parts = [
                parts[m] + parts[m + half] if m + half < len(parts) else parts[m]
                for m in range(half)
            ]
        return parts[0]

    def col_body(j, acc):
        chunk = x_ref[:, pl.ds(j * _CW, _CW)]
        xc = jnp.clip(chunk, -1.0, 1.0)
        return acc + tree128(jnp.exp2(xc * _SE - _SE))

    acc = jax.lax.fori_loop(
        0, n_full, col_body, jnp.zeros((_R, 128), jnp.float32), unroll=2
    )
    s0 = jnp.sum(acc, axis=1)
    if tail:
        xc = jnp.clip(x_ref[:, pl.ds(n_full * _CW, tail)], -1.0, 1.0)
        s0 = s0 + jnp.sum(jnp.exp2(xc * _SE - _SE), axis=1)

    onehot = jax.lax.broadcasted_iota(jnp.int32, (_R, 128), 1) == lane_ref[...]
    c = jnp.sum(jnp.where(onehot, jnp.clip(x128_ref[...], -1.0, 1.0), 0.0), axis=1)

    t_new = _SCALE * (c * _COS_M - _SIN_M * jnp.sqrt(jnp.maximum(1.0 - c * c, 0.0)))
    e_old = jnp.exp2(c * _SE - _SE)
    e_new = jnp.exp(t_new - _SCALE)
    s = s0 - e_old + e_new
    row_loss = _SCALE + jnp.log(s) - t_new

    @pl.when(i == 0)
    def _():
        out_ref[0, 0] = 0.0

    out_ref[0, 0] += jnp.sum(row_loss) * (1.0 / n_rows)


@jax.jit
def kernel(logits, labels):
    n_rows, n_cols = logits.shape
    labels = labels.astype(jnp.int32)

    x128 = _sc_gather_rows(logits, labels)
    lane128 = (labels % 128).reshape(n_rows, 1)

    out = pl.pallas_call(
        functools.partial(_loss_body, n_rows=n_rows, n_cols=n_cols),
        grid=(n_rows // _R,),
        in_specs=[
            pl.BlockSpec((_R, 1), lambda i: (i, 0)),
            pl.BlockSpec((_R, 128), lambda i: (i, 0)),
            pl.BlockSpec((_R, n_cols), lambda i: (i, 0)),
        ],
        out_specs=pl.BlockSpec((1, 1), lambda i: (0, 0), memory_space=pltpu.SMEM),
        out_shape=jax.ShapeDtypeStruct((1, 1), jnp.float32),
    )(lane128, x128, logits)
    return out[0, 0]


# speedup vs baseline: 2.2306x; 1.0398x over previous
import functools
import math

import jax
import jax.numpy as jnp
from jax.experimental import pallas as pl
from jax.experimental.pallas import tpu as pltpu
from jax.experimental.pallas import tpu_sc as plsc

_SCALE = 64.0
_MARGIN = 0.5
_COS_M = math.cos(_MARGIN)
_SIN_M = math.sin(_MARGIN)
_LOG2E = math.log2(math.e)
_SE = _SCALE * _LOG2E

_R = 32
_CW = 2048


def _sc_gather_rows(logits, labels):
    n_rows, n_cols = logits.shape

    @functools.partial(
        pl.kernel,
        out_type=jax.ShapeDtypeStruct((n_rows, 128), logits.dtype),
        mesh=plsc.ScalarSubcoreMesh(axis_name="c", num_cores=2),
        scratch_types=[
            pltpu.SMEM((n_rows,), jnp.int32),
            pltpu.SemaphoreType.DMA,
            pltpu.SemaphoreType.DMA,
        ],
    )
    def gather_kernel(x_hbm, l_hbm, o_hbm, l_smem, sem_l, sem_d):
        core = jax.lax.axis_index("c")
        pltpu.async_copy(l_hbm, l_smem, sem_l).wait()
        half = n_rows // 2
        base = core * half

        @pl.loop(0, half)
        def _(i):
            r = base + i
            st = (l_smem[r] // 128) * 128
            pltpu.async_copy(x_hbm.at[r, pl.ds(st, 128)], o_hbm.at[r], sem_d)

        @pl.loop(0, half)
        def _(i):
            r = base + i
            st = (l_smem[r] // 128) * 128
            pltpu.make_async_copy(
                x_hbm.at[r, pl.ds(st, 128)], o_hbm.at[r], sem_d
            ).wait()

    return gather_kernel(logits, labels)


def _loss_body(lane_ref, x128_ref, x_ref, out_ref, *, n_rows, n_cols):
    i = pl.program_id(0)

    n_full = n_cols // _CW
    tail = n_cols - n_full * _CW

    def tree128(v):
        parts = [v[:, k * 128:(k + 1) * 128] for k in range(v.shape[1] // 128)]
        while len(parts) > 1:
            half = (len(parts) + 1) // 2
            parts = [
                parts[m] + parts[m + half] if m + half < len(parts) else parts[m]
                for m in range(half)
            ]
        return parts[0]

    def col_body(j, acc):
        chunk = x_ref[:, pl.ds(j * _CW, _CW)]
        xc = jnp.clip(chunk, -1.0, 1.0)
        return acc + tree128(jnp.exp2(xc * _SE - _SE))

    acc = jax.lax.fori_loop(
        0, n_full, col_body, jnp.zeros((_R, 128), jnp.float32), unroll=2
    )
    s0 = jnp.sum(acc, axis=1)
    if tail:
        xc = jnp.clip(x_ref[:, pl.ds(n_full * _CW, tail)], -1.0, 1.0)
        s0 = s0 + jnp.sum(jnp.exp2(xc * _SE - _SE), axis=1)

    onehot = jax.lax.broadcasted_iota(jnp.int32, (_R, 128), 1) == lane_ref[...]
    c = jnp.sum(jnp.where(onehot, jnp.clip(x128_ref[...], -1.0, 1.0), 0.0), axis=1)

    t_new = _SCALE * (c * _COS_M - _SIN_M * jnp.sqrt(jnp.maximum(1.0 - c * c, 0.0)))
    e_old = jnp.exp2(c * _SE - _SE)
    e_new = jnp.exp(t_new - _SCALE)
    s = s0 - e_old + e_new
    row_loss = _SCALE + jnp.log(s) - t_new

    @pl.when(i == 0)
    def _():
        out_ref[0, 0] = 0.0

    out_ref[0, 0] += jnp.sum(row_loss) * (1.0 / n_rows)


@jax.jit
def kernel(logits, labels):
    n_rows, n_cols = logits.shape
    labels = labels.astype(jnp.int32)

    x128 = _sc_gather_rows(logits, labels)
    lane128 = (labels % 128).reshape(n_rows, 1)

    out = pl.pallas_call(
        functools.partial(_loss_body, n_rows=n_rows, n_cols=n_cols),
        grid=(n_rows // _R,),
        in_specs=[
            pl.BlockSpec((_R, 1), lambda i: (i, 0)),
            pl.BlockSpec((_R, 128), lambda i: (i, 0)),
            pl.BlockSpec((_R, n_cols), lambda i: (i, 0)),
        ],
        out_specs=pl.BlockSpec((1, 1), lambda i: (0, 0), memory_space=pltpu.SMEM),
        out_shape=jax.ShapeDtypeStruct((1, 1), jnp.float32),
    )(lane128, x128, logits)
    return out[0, 0]


# speedup vs baseline: 2.2307x; 1.0000x over previous
import functools
import math

import jax
import jax.numpy as jnp
from jax.experimental import pallas as pl
from jax.experimental.pallas import tpu as pltpu
from jax.experimental.pallas import tpu_sc as plsc

_SCALE = 64.0
_MARGIN = 0.5
_COS_M = math.cos(_MARGIN)
_SIN_M = math.sin(_MARGIN)
_LOG2E = math.log2(math.e)
_SE = _SCALE * _LOG2E

_R = 64
_CW = 2048


def _sc_gather_rows(logits, labels):
    n_rows, n_cols = logits.shape

    @functools.partial(
        pl.kernel,
        out_type=jax.ShapeDtypeStruct((n_rows, 128), logits.dtype),
        mesh=plsc.ScalarSubcoreMesh(axis_name="c", num_cores=2),
        scratch_types=[
            pltpu.SMEM((n_rows,), jnp.int32),
            pltpu.SemaphoreType.DMA,
            pltpu.SemaphoreType.DMA,
        ],
    )
    def gather_kernel(x_hbm, l_hbm, o_hbm, l_smem, sem_l, sem_d):
        core = jax.lax.axis_index("c")
        pltpu.async_copy(l_hbm, l_smem, sem_l).wait()
        half = n_rows // 2
        base = core * half

        @pl.loop(0, half)
        def _(i):
            r = base + i
            st = (l_smem[r] // 128) * 128
            pltpu.async_copy(x_hbm.at[r, pl.ds(st, 128)], o_hbm.at[r], sem_d)

        @pl.loop(0, half)
        def _(i):
            r = base + i
            st = (l_smem[r] // 128) * 128
            pltpu.make_async_copy(
                x_hbm.at[r, pl.ds(st, 128)], o_hbm.at[r], sem_d
            ).wait()

    return gather_kernel(logits, labels)


def _loss_body(lane_ref, x128_ref, x_ref, out_ref, *, n_rows, n_cols):
    i = pl.program_id(0)

    n_full = n_cols // _CW
    tail = n_cols - n_full * _CW

    def tree128(v):
        parts = [v[:, k * 128:(k + 1) * 128] for k in range(v.shape[1] // 128)]
        while len(parts) > 1:
            half = (len(parts) + 1) // 2
            parts = [
                parts[m] + parts[m + half] if m + half < len(parts) else parts[m]
                for m in range(half)
            ]
        return parts[0]

    def col_body(j, acc):
        chunk = x_ref[:, pl.ds(j * _CW, _CW)]
        xc = jnp.clip(chunk, -1.0, 1.0)
        return acc + tree128(jnp.exp2(xc * _SE - _SE))

    acc = jax.lax.fori_loop(
        0, n_full, col_body, jnp.zeros((_R, 128), jnp.float32), unroll=2
    )
    s0 = jnp.sum(acc, axis=1)
    if tail:
        xc = jnp.clip(x_ref[:, pl.ds(n_full * _CW, tail)], -1.0, 1.0)
        s0 = s0 + jnp.sum(jnp.exp2(xc * _SE - _SE), axis=1)

    onehot = jax.lax.broadcasted_iota(jnp.int32, (_R, 128), 1) == lane_ref[...]
    c = jnp.sum(jnp.where(onehot, jnp.clip(x128_ref[...], -1.0, 1.0), 0.0), axis=1)

    t_new = _SCALE * (c * _COS_M - _SIN_M * jnp.sqrt(jnp.maximum(1.0 - c * c, 0.0)))
    e_old = jnp.exp2(c * _SE - _SE)
    e_new = jnp.exp(t_new - _SCALE)
    s = s0 - e_old + e_new
    row_loss = _SCALE + jnp.log(s) - t_new

    @pl.when(i == 0)
    def _():
        out_ref[0, 0] = 0.0

    out_ref[0, 0] += jnp.sum(row_loss) * (1.0 / n_rows)


@jax.jit
def kernel(logits, labels):
    n_rows, n_cols = logits.shape
    labels = labels.astype(jnp.int32)

    x128 = _sc_gather_rows(logits, labels)
    lane128 = (labels % 128).reshape(n_rows, 1)

    out = pl.pallas_call(
        functools.partial(_loss_body, n_rows=n_rows, n_cols=n_cols),
        grid=(n_rows // _R,),
        in_specs=[
            pl.BlockSpec((_R, 1), lambda i: (i, 0)),
            pl.BlockSpec((_R, 128), lambda i: (i, 0)),
            pl.BlockSpec((_R, n_cols), lambda i: (i, 0)),
        ],
        out_specs=pl.BlockSpec((1, 1), lambda i: (0, 0), memory_space=pltpu.SMEM),
        out_shape=jax.ShapeDtypeStruct((1, 1), jnp.float32),
    )(lane128, x128, logits)
    return out[0, 0]


# speedup vs baseline: 2.2378x; 1.0032x over previous
import functools
import math

import jax
import jax.numpy as jnp
from jax.experimental import pallas as pl
from jax.experimental.pallas import tpu as pltpu
from jax.experimental.pallas import tpu_sc as plsc

_SCALE = 64.0
_MARGIN = 0.5
_COS_M = math.cos(_MARGIN)
_SIN_M = math.sin(_MARGIN)
_LOG2E = math.log2(math.e)
_SE = _SCALE * _LOG2E

_R = 32
_CW = 2048


def _sc_gather_rows(logits, labels):
    n_rows, n_cols = logits.shape

    @functools.partial(
        pl.kernel,
        out_type=jax.ShapeDtypeStruct((n_rows, 128), logits.dtype),
        mesh=plsc.ScalarSubcoreMesh(axis_name="c", num_cores=2),
        scratch_types=[
            pltpu.SMEM((n_rows,), jnp.int32),
            pltpu.SemaphoreType.DMA,
            pltpu.SemaphoreType.DMA,
        ],
    )
    def gather_kernel(x_hbm, l_hbm, o_hbm, l_smem, sem_l, sem_d):
        core = jax.lax.axis_index("c")
        pltpu.async_copy(l_hbm, l_smem, sem_l).wait()
        half = n_rows // 2
        base = core * half

        @pl.loop(0, half)
        def _(i):
            r = base + i
            st = (l_smem[r] // 128) * 128
            pltpu.async_copy(x_hbm.at[r, pl.ds(st, 128)], o_hbm.at[r], sem_d)

        @pl.loop(0, half)
        def _(i):
            r = base + i
            st = (l_smem[r] // 128) * 128
            pltpu.make_async_copy(
                x_hbm.at[r, pl.ds(st, 128)], o_hbm.at[r], sem_d
            ).wait()

    return gather_kernel(logits, labels)


def _loss_body(lane_ref, x128_ref, x_ref, out_ref, *, n_rows, n_cols):
    i = pl.program_id(0)

    n_full = n_cols // _CW
    tail = n_cols - n_full * _CW

    def tree128(v):
        parts = [v[:, k * 128:(k + 1) * 128] for k in range(v.shape[1] // 128)]
        while len(parts) > 1:
            half = (len(parts) + 1) // 2
            parts = [
                parts[m] + parts[m + half] if m + half < len(parts) else parts[m]
                for m in range(half)
            ]
        return parts[0]

    def col_body(j, acc):
        chunk = x_ref[:, pl.ds(j * _CW, _CW)]
        xc = jnp.clip(chunk, -1.0, 1.0)
        return acc + tree128(jnp.exp2(xc * _SE - _SE))

    acc = jax.lax.fori_loop(
        0, n_full, col_body, jnp.zeros((_R, 128), jnp.float32), unroll=4
    )
    s0 = jnp.sum(acc, axis=1)
    if tail:
        xc = jnp.clip(x_ref[:, pl.ds(n_full * _CW, tail)], -1.0, 1.0)
        s0 = s0 + jnp.sum(jnp.exp2(xc * _SE - _SE), axis=1)

    onehot = jax.lax.broadcasted_iota(jnp.int32, (_R, 128), 1) == lane_ref[...]
    c = jnp.sum(jnp.where(onehot, jnp.clip(x128_ref[...], -1.0, 1.0), 0.0), axis=1)

    t_new = _SCALE * (c * _COS_M - _SIN_M * jnp.sqrt(jnp.maximum(1.0 - c * c, 0.0)))
    e_old = jnp.exp2(c * _SE - _SE)
    e_new = jnp.exp(t_new - _SCALE)
    s = s0 - e_old + e_new
    row_loss = _SCALE + jnp.log(s) - t_new

    @pl.when(i == 0)
    def _():
        out_ref[0, 0] = 0.0

    out_ref[0, 0] += jnp.sum(row_loss) * (1.0 / n_rows)


@jax.jit
def kernel(logits, labels):
    n_rows, n_cols = logits.shape
    labels = labels.astype(jnp.int32)

    x128 = _sc_gather_rows(logits, labels)
    lane128 = (labels % 128).reshape(n_rows, 1)

    out = pl.pallas_call(
        functools.partial(_loss_body, n_rows=n_rows, n_cols=n_cols),
        grid=(n_rows // _R,),
        in_specs=[
            pl.BlockSpec((_R, 1), lambda i: (i, 0)),
            pl.BlockSpec((_R, 128), lambda i: (i, 0)),
            pl.BlockSpec((_R, n_cols), lambda i: (i, 0)),
        ],
        out_specs=pl.BlockSpec((1, 1), lambda i: (0, 0), memory_space=pltpu.SMEM),
        out_shape=jax.ShapeDtypeStruct((1, 1), jnp.float32),
    )(lane128, x128, logits)
    return out[0, 0]
